# baseline (speedup 1.0000x reference)
import jax
import jax.numpy as jnp
from jax.experimental import pallas as pl
from jax.experimental.pallas import tpu as pltpu

_K = 256
_ROWS = 64
_COLS = 8192
_BLOCK_ROWS = 64


def _sparsify_kernel(x_ref, o_ref):
    x = x_ref[...]
    i = jax.lax.bitcast_convert_type(x, jnp.int32)
    keys = jnp.where(i >= 0, i, i ^ jnp.int32(0x7FFFFFFF))
    int_min = jnp.int32(-2147483648)

    kf = jnp.float32(_K)

    xb = jax.lax.bitcast_convert_type(
        jax.lax.shift_right_logical(i, 16).astype(jnp.int16), jnp.bfloat16)
    ihi = jax.lax.shift_right_arithmetic(i, 16)

    def bf16_count(tbf):
        t = jnp.where(xb >= tbf, jnp.bfloat16(1), jnp.bfloat16(0))
        w = _COLS
        while w > 32:
            w //= 2
            t = t[:, :w] + t[:, w:]
        return jnp.sum(t.astype(jnp.float32), axis=1, keepdims=True)

    neg0 = jnp.where(ihi == jnp.int32(-32768), jnp.float32(1),
                     jnp.float32(0))
    c1 = bf16_count(jnp.bfloat16(0.0)) - jnp.sum(neg0, axis=1,
                                                 keepdims=True)
    tbh = jnp.where(c1 >= kf, jnp.int32(0x8000), jnp.int32(0))

    def hi_round(j, tbh):
        trial = tbh | (jnp.int32(1) << (jnp.int32(15) - j))
        k16 = trial ^ jnp.int32(0x8000)
        fb = jnp.where(k16 < 32768, k16, k16 ^ jnp.int32(0x7FFF))
        tbf = jax.lax.bitcast_convert_type(fb.astype(jnp.int16),
                                           jnp.bfloat16)
        return jnp.where(bf16_count(tbf) >= kf, trial, tbh)

    for j in range(1, 16):
        tbh = hi_round(j, tbh)

    def one_round(j, tb):
        bit = jnp.int32(1) << (jnp.int32(31) - j)
        trial = tb | bit
        thresh = trial ^ int_min
        t = jnp.where(keys >= thresh, jnp.float32(1), jnp.float32(0))
        cnt = jnp.sum(t, axis=1, keepdims=True)
        return jnp.where(cnt >= kf, trial, tb)

    tb = jax.lax.shift_left(tbh, 16)
    for j in range(16, 32):
        tb = one_round(j, tb)
    tkey = tb ^ int_min
    o_ref[...] = jnp.where(keys >= tkey, x, jnp.float32(0.0))


def kernel(x):
    grid = (_ROWS // _BLOCK_ROWS,)
    return pl.pallas_call(
        _sparsify_kernel,
        grid=grid,
        in_specs=[pl.BlockSpec((_BLOCK_ROWS, _COLS), lambda i: (i, 0))],
        out_specs=pl.BlockSpec((_BLOCK_ROWS, _COLS), lambda i: (i, 0)),
        out_shape=jax.ShapeDtypeStruct((_ROWS, _COLS), jnp.float32),
    )(x)


# speedup vs baseline: 1.0642x; 1.0642x over previous
import jax
import jax.numpy as jnp
from jax.experimental import pallas as pl
from jax.experimental.pallas import tpu as pltpu

_K = 256
_ROWS = 64
_COLS = 8192
_BLOCK_ROWS = 64


def _sparsify_kernel(x_ref, o_ref):
    x = x_ref[...]
    i = jax.lax.bitcast_convert_type(x, jnp.int32)
    keys = jnp.where(i >= 0, i, i ^ jnp.int32(0x7FFFFFFF))
    int_min = jnp.int32(-2147483648)

    def one_round(j, tb):
        bit = jnp.int32(1) << (jnp.int32(31) - j)
        trial = tb | bit
        thresh = trial ^ int_min
        t = jnp.where(keys >= thresh, jnp.float32(1), jnp.float32(0))
        cnt = jnp.sum(t, axis=1, keepdims=True)
        return jnp.where(cnt >= jnp.float32(_K), trial, tb)

    tb = jnp.zeros((x.shape[0], 1), jnp.int32)
    for j in range(32):
        tb = one_round(j, tb)
    tkey = tb ^ int_min
    o_ref[...] = jnp.where(keys >= tkey, x, jnp.float32(0.0))


def kernel(x):
    grid = (_ROWS // _BLOCK_ROWS,)
    return pl.pallas_call(
        _sparsify_kernel,
        grid=grid,
        in_specs=[pl.BlockSpec((_BLOCK_ROWS, _COLS), lambda i: (i, 0))],
        out_specs=pl.BlockSpec((_BLOCK_ROWS, _COLS), lambda i: (i, 0)),
        out_shape=jax.ShapeDtypeStruct((_ROWS, _COLS), jnp.float32),
    )(x)
